# baseline (speedup 1.0000x reference)
import functools

import jax
import jax.numpy as jnp
from jax import lax
from jax.experimental import pallas as pl
from jax.experimental.pallas import tpu as pltpu
from jax.experimental.pallas import tpu_sc as plsc

N = 10000
NP = 10240
E = 160000
D_IN = 256
D_HID = 512
D_OUT = 256

F = 128
K = 128
G = 4
BM = 320
NBLK = NP // BM
ROWS_PER_TILE = NP // 16
EP = 163840
EBP = EP // K
BPT = EBP // 16
NG = BPT // G

_MESH = plsc.VectorSubcoreMesh(core_axis_name="c", subcore_axis_name="s")


@functools.partial(
    pl.kernel,
    out_type=jax.ShapeDtypeStruct((2 * NP, F), jnp.float32),
    mesh=_MESH,
    scratch_types=[
        pltpu.VMEM((K,), jnp.int32),
        pltpu.VMEM((K, F), jnp.float32),
        pltpu.VMEM_SHARED((NP, F), jnp.float32),
    ],
)
def _hist(dst_hbm, ones_hbm, zeros_hbm, out_hbm, dstv, onesv, hist_sh):
    cid = lax.axis_index("c")
    sid = lax.axis_index("s")
    wid = cid * 16 + sid
    pltpu.sync_copy(ones_hbm, onesv)
    pltpu.sync_copy(zeros_hbm, hist_sh.at[pl.ds(sid * ROWS_PER_TILE, ROWS_PER_TILE)])
    plsc.subcore_barrier()
    first = wid * (EBP // 32)

    def body(b, carry):
        off = (first + b) * K
        pltpu.sync_copy(dst_hbm.at[pl.ds(off, K)], dstv)
        pltpu.sync_copy(onesv, hist_sh.at[dstv], add=True)
        return carry

    lax.fori_loop(0, EBP // 32, body, 0)
    plsc.subcore_barrier()
    r0 = sid * ROWS_PER_TILE
    pltpu.sync_copy(hist_sh.at[pl.ds(r0, ROWS_PER_TILE)],
                    out_hbm.at[pl.ds(cid * NP + r0, ROWS_PER_TILE)])


def _make_scatter(C):
    assert C == 2

    @functools.partial(
        pl.kernel,
        out_type=jax.ShapeDtypeStruct((C * NP, F), jnp.float32),
        mesh=_MESH,
        scratch_types=[
            pltpu.VMEM((K,), jnp.int32),
            pltpu.VMEM((K,), jnp.int32),
            pltpu.VMEM((K, F), jnp.float32),
            pltpu.VMEM_SHARED((NP, F), jnp.float32),
            pltpu.SemaphoreType.DMA,
        ],
    )
    def _scatter(h_hbm, srcadj_hbm, dst_hbm, zeros_hbm, dep_hbm, out_hbm,
                 srcv, dstv, rows, acc_sh, sem):
        del dep_hbm
        cid = lax.axis_index("c")
        sid = lax.axis_index("s")
        r0 = sid * ROWS_PER_TILE
        sbase = cid * EP + sid * BPT * K
        dbase = sid * BPT * K

        pltpu.sync_copy(zeros_hbm, acc_sh.at[pl.ds(r0, ROWS_PER_TILE)])
        plsc.subcore_barrier()

        def body(b, carry):
            pltpu.sync_copy(srcadj_hbm.at[pl.ds(sbase + b * K, K)], srcv)
            pltpu.sync_copy(dst_hbm.at[pl.ds(dbase + b * K, K)], dstv)
            pltpu.async_copy(h_hbm.at[srcv], rows, sem).wait()
            pltpu.sync_copy(rows, acc_sh.at[dstv], add=True)
            return carry

        lax.fori_loop(0, BPT, body, 0)
        plsc.subcore_barrier()
        pltpu.sync_copy(acc_sh.at[pl.ds(r0, ROWS_PER_TILE)],
                        out_hbm.at[pl.ds(cid * NP + r0, ROWS_PER_TILE)])
        plsc.subcore_barrier()

    return _scatter


_scatter2 = _make_scatter(2)


def _m1_body(x_ref, w_ref, dinv_ref, o_ref):
    xs = x_ref[...] * dinv_ref[...]
    o_ref[...] = jnp.dot(xs, w_ref[...], preferred_element_type=jnp.float32)


def _m1(x, Wh, dinv2d):
    CH = 2
    return pl.pallas_call(
        _m1_body,
        grid=(CH, NBLK),
        in_specs=[
            pl.BlockSpec((BM, D_IN), lambda c, i: (i, 0)),
            pl.BlockSpec((D_IN, F), lambda c, i: (0, c)),
            pl.BlockSpec((BM, 1), lambda c, i: (i, 0)),
        ],
        out_specs=pl.BlockSpec((BM, F), lambda c, i: (c * NBLK + i, 0)),
        out_shape=jax.ShapeDtypeStruct((CH * NP, F), jnp.float32),
    )(x, Wh, dinv2d)


def _m2_first_body(acc_ref, h_ref, dinv_ref, b_ref, w_ref, o_ref):
    k = pl.program_id(1)
    dinv = dinv_ref[...]
    z = jax.nn.relu(dinv * (acc_ref[...] + h_ref[...]) + b_ref[0])
    p = jnp.dot(dinv * z, w_ref[...], preferred_element_type=jnp.float32)

    @pl.when(k == 0)
    def _():
        o_ref[...] = p

    @pl.when(k > 0)
    def _():
        o_ref[...] += p


def _m2_second_body(acc_ref, h_ref, dinv_ref, b_ref, w_ref, prev_ref, o_ref):
    k = pl.program_id(1)
    dinv = dinv_ref[...]
    z = jax.nn.relu(dinv * (acc_ref[...] + h_ref[...]) + b_ref[0])
    p = jnp.dot(dinv * z, w_ref[...], preferred_element_type=jnp.float32)

    @pl.when(k == 0)
    def _():
        o_ref[...] = prev_ref[...] + p

    @pl.when(k > 0)
    def _():
        o_ref[...] += p


def _m2(acc_half, h_half, dinv2d, b_half, W_half, prev=None):
    specs = [
        pl.BlockSpec((BM, F), lambda i, k: (k * NBLK + i, 0)),
        pl.BlockSpec((BM, F), lambda i, k: (k * NBLK + i, 0)),
        pl.BlockSpec((BM, 1), lambda i, k: (i, 0)),
        pl.BlockSpec((1, 1, F), lambda i, k: (k, 0, 0)),
        pl.BlockSpec((F, D_OUT), lambda i, k: (k, 0)),
    ]
    args = [acc_half, h_half, dinv2d, b_half, W_half]
    body = _m2_first_body
    if prev is not None:
        specs.append(pl.BlockSpec((BM, D_OUT), lambda i, k: (i, 0)))
        args.append(prev)
        body = _m2_second_body
    return pl.pallas_call(
        body,
        grid=(NBLK, 2),
        in_specs=specs,
        out_specs=pl.BlockSpec((BM, D_OUT), lambda i, k: (i, 0)),
        out_shape=jax.ShapeDtypeStruct((NP, D_OUT), jnp.float32),
    )(*args)


def _m3_body(acc_ref, h_ref, dinv_ref, b_ref, o_ref):
    o_ref[...] = dinv_ref[...] * (acc_ref[...] + h_ref[...]) + b_ref[0]


def _m3(acc2, h2c, dinv2d, b2r):
    CH = D_OUT // F
    return pl.pallas_call(
        _m3_body,
        grid=(NBLK, CH),
        in_specs=[
            pl.BlockSpec((BM, F), lambda i, c: (c * NBLK + i, 0)),
            pl.BlockSpec((BM, F), lambda i, c: (c * NBLK + i, 0)),
            pl.BlockSpec((BM, 1), lambda i, c: (i, 0)),
            pl.BlockSpec((1, 1, F), lambda i, c: (c, 0, 0)),
        ],
        out_specs=pl.BlockSpec((BM, F), lambda i, c: (i, c)),
        out_shape=jax.ShapeDtypeStruct((NP, D_OUT), jnp.float32),
    )(acc2, h2c, dinv2d, b2r)


def kernel(x_seq, edge_index_seq, W1, b1, W2, b2):
    x = jnp.pad(x_seq[-1], ((0, NP - N), (0, 0)))
    src = edge_index_seq[-1, 0]
    dst = edge_index_seq[-1, 1]

    onesF = jnp.ones((K, F), jnp.float32)
    zerosF = jnp.zeros((ROWS_PER_TILE, F), jnp.float32)

    padv = jnp.full((EP - E,), NP - 1, jnp.int32)
    srcp = jnp.concatenate([src, padv])
    dstp = jnp.concatenate([dst, padv])

    part = _hist(dstp, onesF, zerosF)
    cnt = part[:NP, 0] + part[NP:, 0]
    dinv = lax.rsqrt(cnt + 1.0)
    dinv2d = dinv[:, None]

    srcadj = (srcp[None, :] + (jnp.arange(2, dtype=jnp.int32) * NP)[:, None]
              ).reshape(-1)

    h1a = _m1(x, W1[:, : 2 * F], dinv2d)
    h1b = _m1(x, W1[:, 2 * F :], dinv2d)
    acc1a = _scatter2(h1a, srcadj, dstp, zerosF, part)
    acc1b = _scatter2(h1b, srcadj, dstp, zerosF, acc1a)
    h2s = _m2(acc1a, h1a, dinv2d, b1[: 2 * F].reshape(2, 1, F), W2[: 2 * F])
    h2s = _m2(acc1b, h1b, dinv2d, b1[2 * F :].reshape(2, 1, F), W2[2 * F :],
              prev=h2s)
    h2c = h2s.reshape(NP, 2, F).transpose(1, 0, 2).reshape(2 * NP, F)
    acc2 = _scatter2(h2c, srcadj, dstp, zerosF, acc1b)
    return _m3(acc2, h2c, dinv2d, b2.reshape(2, 1, F))[:N]


# speedup vs baseline: 1.4562x; 1.4562x over previous
import functools

import jax
import jax.numpy as jnp
from jax import lax
from jax.experimental import pallas as pl
from jax.experimental.pallas import tpu as pltpu
from jax.experimental.pallas import tpu_sc as plsc

N = 10000
NP = 10240
E = 160000
D_IN = 256
D_HID = 512
D_OUT = 256

F = 128
K = 128
BM = 320
NBLK = NP // BM
ROWS_PER_TILE = NP // 16
EB = E // K

_MESH = plsc.VectorSubcoreMesh(core_axis_name="c", subcore_axis_name="s")


def _split(nworkers, wid):
    per = EB // nworkers
    extra = EB - per * nworkers
    first = wid * per + jnp.minimum(wid, extra)
    num = per + jnp.where(wid < extra, 1, 0)
    return first, num


@functools.partial(
    pl.kernel,
    out_type=jax.ShapeDtypeStruct((2 * NP, F), jnp.float32),
    mesh=_MESH,
    scratch_types=[
        pltpu.VMEM((K,), jnp.int32),
        pltpu.VMEM((K, F), jnp.float32),
        pltpu.VMEM_SHARED((NP, F), jnp.float32),
    ],
)
def _hist(dst_hbm, ones_hbm, zeros_hbm, out_hbm, dstv, onesv, hist_sh):
    cid = lax.axis_index("c")
    sid = lax.axis_index("s")
    wid = cid * 16 + sid
    pltpu.sync_copy(ones_hbm, onesv)
    pltpu.sync_copy(zeros_hbm, hist_sh.at[pl.ds(sid * ROWS_PER_TILE, ROWS_PER_TILE)])
    plsc.subcore_barrier()
    first, num = _split(32, wid)

    def body(b, carry):
        off = (first + b) * K
        pltpu.sync_copy(dst_hbm.at[pl.ds(off, K)], dstv)
        pltpu.sync_copy(onesv, hist_sh.at[dstv], add=True)
        return carry

    lax.fori_loop(0, num, body, 0)
    plsc.subcore_barrier()
    r0 = sid * ROWS_PER_TILE
    pltpu.sync_copy(hist_sh.at[pl.ds(r0, ROWS_PER_TILE)],
                    out_hbm.at[pl.ds(cid * NP + r0, ROWS_PER_TILE)])


def _make_scatter(C):
    CPC = C // 2

    @functools.partial(
        pl.kernel,
        out_type=jax.ShapeDtypeStruct((C * NP, F), jnp.float32),
        mesh=_MESH,
        scratch_types=[
            pltpu.VMEM((K,), jnp.int32),
            pltpu.VMEM((K,), jnp.int32),
            pltpu.VMEM((K, F), jnp.float32),
            pltpu.VMEM_SHARED((NP, F), jnp.float32),
            pltpu.SemaphoreType.DMA,
        ],
    )
    def _scatter(h_hbm, srcadj_hbm, dst_hbm, zeros_hbm, out_hbm,
                 srcv, dstv, rows, acc_sh, sem):
        cid = lax.axis_index("c")
        sid = lax.axis_index("s")
        first, num = _split(16, sid)
        r0 = sid * ROWS_PER_TILE
        for cc in range(CPC):
            c = cid * CPC + cc
            pltpu.sync_copy(zeros_hbm, acc_sh.at[pl.ds(r0, ROWS_PER_TILE)])
            plsc.subcore_barrier()

            def body(b, carry):
                eoff = (first + b) * K
                pltpu.sync_copy(srcadj_hbm.at[pl.ds(c * E + eoff, K)], srcv)
                pltpu.sync_copy(dst_hbm.at[pl.ds(eoff, K)], dstv)
                pltpu.async_copy(h_hbm.at[srcv], rows, sem).wait()
                pltpu.sync_copy(rows, acc_sh.at[dstv], add=True)
                return carry

            lax.fori_loop(0, num, body, 0)
            plsc.subcore_barrier()
            pltpu.sync_copy(acc_sh.at[pl.ds(r0, ROWS_PER_TILE)],
                            out_hbm.at[pl.ds(c * NP + r0, ROWS_PER_TILE)])
            plsc.subcore_barrier()

    return _scatter


_scatter4 = _make_scatter(4)
_scatter2 = _make_scatter(2)


def _m1_body(x_ref, w_ref, dinv_ref, o_ref):
    xs = x_ref[...] * dinv_ref[...]
    o_ref[...] = jnp.dot(xs, w_ref[...], preferred_element_type=jnp.float32)


def _m1(x, W1, dinv2d):
    CH = D_HID // F
    return pl.pallas_call(
        _m1_body,
        grid=(CH, NBLK),
        in_specs=[
            pl.BlockSpec((BM, D_IN), lambda c, i: (i, 0)),
            pl.BlockSpec((D_IN, F), lambda c, i: (0, c)),
            pl.BlockSpec((BM, 1), lambda c, i: (i, 0)),
        ],
        out_specs=pl.BlockSpec((BM, F), lambda c, i: (c * NBLK + i, 0)),
        out_shape=jax.ShapeDtypeStruct((CH * NP, F), jnp.float32),
    )(x, W1, dinv2d)


def _m2_body(acc_ref, h_ref, dinv_ref, b_ref, w_ref, o_ref):
    k = pl.program_id(1)
    dinv = dinv_ref[...]
    z = jax.nn.relu(dinv * (acc_ref[...] + h_ref[...]) + b_ref[0])
    p = jnp.dot(dinv * z, w_ref[...], preferred_element_type=jnp.float32)

    @pl.when(k == 0)
    def _():
        o_ref[...] = p

    @pl.when(k > 0)
    def _():
        o_ref[...] += p


def _m2(acc1, h1s, dinv2d, b1r, W2):
    KC = D_HID // F
    return pl.pallas_call(
        _m2_body,
        grid=(NBLK, KC),
        in_specs=[
            pl.BlockSpec((BM, F), lambda i, k: (k * NBLK + i, 0)),
            pl.BlockSpec((BM, F), lambda i, k: (k * NBLK + i, 0)),
            pl.BlockSpec((BM, 1), lambda i, k: (i, 0)),
            pl.BlockSpec((1, 1, F), lambda i, k: (k, 0, 0)),
            pl.BlockSpec((F, D_OUT), lambda i, k: (k, 0)),
        ],
        out_specs=pl.BlockSpec((BM, D_OUT), lambda i, k: (i, 0)),
        out_shape=jax.ShapeDtypeStruct((NP, D_OUT), jnp.float32),
    )(acc1, h1s, dinv2d, b1r, W2)


def _m3_body(acc_ref, h_ref, dinv_ref, b_ref, o_ref):
    o_ref[...] = dinv_ref[...] * (acc_ref[...] + h_ref[...]) + b_ref[0]


def _m3(acc2, h2c, dinv2d, b2r):
    CH = D_OUT // F
    return pl.pallas_call(
        _m3_body,
        grid=(NBLK, CH),
        in_specs=[
            pl.BlockSpec((BM, F), lambda i, c: (c * NBLK + i, 0)),
            pl.BlockSpec((BM, F), lambda i, c: (c * NBLK + i, 0)),
            pl.BlockSpec((BM, 1), lambda i, c: (i, 0)),
            pl.BlockSpec((1, 1, F), lambda i, c: (c, 0, 0)),
        ],
        out_specs=pl.BlockSpec((BM, F), lambda i, c: (i, c)),
        out_shape=jax.ShapeDtypeStruct((NP, D_OUT), jnp.float32),
    )(acc2, h2c, dinv2d, b2r)


def kernel(x_seq, edge_index_seq, W1, b1, W2, b2):
    x = jnp.pad(x_seq[-1], ((0, NP - N), (0, 0)))
    src = edge_index_seq[-1, 0]
    dst = edge_index_seq[-1, 1]

    onesF = jnp.ones((K, F), jnp.float32)
    zerosF = jnp.zeros((ROWS_PER_TILE, F), jnp.float32)

    part = _hist(dst, onesF, zerosF)
    cnt = part[:NP, 0] + part[NP:, 0]
    dinv = lax.rsqrt(cnt + 1.0)
    dinv2d = dinv[:, None]

    srcadj4 = (src[None, :] + (jnp.arange(4, dtype=jnp.int32) * NP)[:, None]
               ).reshape(-1)

    h1s = _m1(x, W1, dinv2d)
    acc1 = _scatter4(h1s, srcadj4, dst, zerosF)
    h2s = _m2(acc1, h1s, dinv2d, b1.reshape(4, 1, F), W2)
    h2c = h2s.reshape(NP, 2, F).transpose(1, 0, 2).reshape(2 * NP, F)
    acc2 = _scatter2(h2c, srcadj4[: 2 * E], dst, zerosF)
    return _m3(acc2, h2c, dinv2d, b2.reshape(2, 1, F))[:N]
